# baseline (speedup 1.0000x reference)
import functools

import jax
import jax.numpy as jnp
from jax import lax
from jax.experimental import pallas as pl
from jax.experimental.pallas import tpu as pltpu
from jax.experimental.pallas import tpu_sc as plsc

EMBED_DIM = 256
N_AG = 10000
N_TASK = 10000
N_NODES = N_AG + N_TASK
DEG = 16
N_EDGES = N_AG * DEG
FIN_TASK_TYPE = 3
NEG_SLOPE = 0.01

_NC = 2
_NS = 16
_NW = _NC * _NS
_AG_PER = 320
_EDGE_PER = _AG_PER * DEG

_TC_ROWS = 2048


def _tc_project_body(nf_ref, w_ref, t_ref, a_ref, b_ref):
    x = nf_ref[...]
    w = w_ref[...]
    p = lax.dot_general(
        w, x, dimension_numbers=(((1,), (1,)), ((), ())),
        preferred_element_type=jnp.float32)
    fin = t_ref[...] == FIN_TASK_TYPE
    a_ref[...] = jnp.where(fin, -jnp.inf, p[0, :])
    b_ref[...] = p[1, :]


def _tc_project(nf, node_type, w2):
    return pl.pallas_call(
        _tc_project_body,
        grid=(pl.cdiv(N_NODES, _TC_ROWS),),
        in_specs=[
            pl.BlockSpec((_TC_ROWS, EMBED_DIM), lambda i: (i, 0)),
            pl.BlockSpec((2, EMBED_DIM), lambda i: (0, 0)),
            pl.BlockSpec((_TC_ROWS,), lambda i: (i,)),
        ],
        out_specs=[
            pl.BlockSpec((_TC_ROWS,), lambda i: (i,)),
            pl.BlockSpec((_TC_ROWS,), lambda i: (i,)),
        ],
        out_shape=[
            jax.ShapeDtypeStruct((N_NODES,), jnp.float32),
            jax.ShapeDtypeStruct((N_NODES,), jnp.float32),
        ],
    )(nf, w2, node_type)


def _sc_softmax_body(a_hbm, b_hbm, ei_hbm, out_hbm, a_v, b_v, idx_v, out_v):
    wid = lax.axis_index("s") * _NC + lax.axis_index("c")
    base = jnp.minimum(wid * _AG_PER, N_AG - _AG_PER)
    ebase = base * DEG
    pltpu.sync_copy(a_hbm.at[pl.ds(N_AG, N_TASK)], a_v)
    pltpu.sync_copy(b_hbm.at[pl.ds(base, _AG_PER)], b_v.at[pl.ds(0, _AG_PER)])
    pltpu.sync_copy(ei_hbm.at[0].at[pl.ds(ebase, _EDGE_PER)], idx_v)

    @plsc.parallel_loop(0, _AG_PER, unroll=4)
    def _(i):
        idx = idx_v[pl.ds(i * DEG, DEG)] - N_AG
        av = plsc.load_gather(a_v, [idx])
        bg = b_v[pl.ds(i, DEG)][0]
        s = av + bg
        s = jnp.where(s >= 0, s, s * NEG_SLOPE)
        m = jnp.max(s)
        e = jnp.exp(s - m)
        out_v[pl.ds(i * DEG, DEG)] = e / jnp.sum(e)

    pltpu.sync_copy(out_v, out_hbm.at[pl.ds(ebase, _EDGE_PER)])


@functools.cache
def _sc_softmax_kernel():
    return pl.kernel(
        _sc_softmax_body,
        out_type=jax.ShapeDtypeStruct((N_EDGES,), jnp.float32),
        mesh=plsc.VectorSubcoreMesh(
            core_axis_name="c", subcore_axis_name="s",
            num_cores=_NC, num_subcores=_NS),
        scratch_types=[
            pltpu.VMEM((N_TASK,), jnp.float32),
            pltpu.VMEM((_AG_PER + DEG,), jnp.float32),
            pltpu.VMEM((_EDGE_PER,), jnp.int32),
            pltpu.VMEM((_EDGE_PER,), jnp.float32),
        ],
        compiler_params=pltpu.CompilerParams(needs_layout_passes=False, skip_device_barrier=True, disable_semaphore_checks=True, disable_bounds_checks=True),
    )


def kernel(nf, edge_index, node_type, W_att):
    w2 = W_att.reshape(2, EMBED_DIM)
    a_full, b_full = _tc_project(nf, node_type, w2)
    flat = _sc_softmax_kernel()(a_full, b_full, edge_index)
    return flat.reshape(N_AG, DEG)


# speedup vs baseline: 1.0605x; 1.0605x over previous
import functools

import jax
import jax.numpy as jnp
from jax import lax
from jax.experimental import pallas as pl
from jax.experimental.pallas import tpu as pltpu
from jax.experimental.pallas import tpu_sc as plsc

EMBED_DIM = 256
N_AG = 10000
N_TASK = 10000
N_NODES = N_AG + N_TASK
DEG = 16
N_EDGES = N_AG * DEG
FIN_TASK_TYPE = 3
NEG_SLOPE = 0.01

_NC = 2
_NS = 16
_NW = _NC * _NS
_AG_PER = 320
_EDGE_PER = _AG_PER * DEG

_TC_ROWS = 4096


def _tc_project_body(nf_ref, w_ref, t_ref, a_ref, b_ref):
    x = nf_ref[...]
    w = w_ref[...]
    p = lax.dot_general(
        w, x, dimension_numbers=(((1,), (1,)), ((), ())),
        preferred_element_type=jnp.float32)
    fin = t_ref[...] == FIN_TASK_TYPE
    a_ref[...] = jnp.where(fin, -jnp.inf, p[0, :])
    b_ref[...] = p[1, :]


def _tc_project(nf, node_type, w2):
    return pl.pallas_call(
        _tc_project_body,
        grid=(pl.cdiv(N_NODES, _TC_ROWS),),
        in_specs=[
            pl.BlockSpec((_TC_ROWS, EMBED_DIM), lambda i: (i, 0)),
            pl.BlockSpec((2, EMBED_DIM), lambda i: (0, 0)),
            pl.BlockSpec((_TC_ROWS,), lambda i: (i,)),
        ],
        out_specs=[
            pl.BlockSpec((_TC_ROWS,), lambda i: (i,)),
            pl.BlockSpec((_TC_ROWS,), lambda i: (i,)),
        ],
        out_shape=[
            jax.ShapeDtypeStruct((N_NODES,), jnp.float32),
            jax.ShapeDtypeStruct((N_NODES,), jnp.float32),
        ],
    )(nf, w2, node_type)


def _sc_softmax_body(a_hbm, b_hbm, ei_hbm, out_hbm, a_v, b_v, idx_v, out_v):
    wid = lax.axis_index("s") * _NC + lax.axis_index("c")
    base = jnp.minimum(wid * _AG_PER, N_AG - _AG_PER)
    ebase = base * DEG
    pltpu.sync_copy(a_hbm.at[pl.ds(N_AG, N_TASK)], a_v)
    pltpu.sync_copy(b_hbm.at[pl.ds(base, _AG_PER)], b_v.at[pl.ds(0, _AG_PER)])
    pltpu.sync_copy(ei_hbm.at[0].at[pl.ds(ebase, _EDGE_PER)], idx_v)

    @plsc.parallel_loop(0, _AG_PER, unroll=4)
    def _(i):
        idx = idx_v[pl.ds(i * DEG, DEG)] - N_AG
        av = plsc.load_gather(a_v, [idx])
        bg = b_v[pl.ds(i, DEG)][0]
        s = av + bg
        s = jnp.where(s >= 0, s, s * NEG_SLOPE)
        m = jnp.max(s)
        e = jnp.exp(s - m)
        out_v[pl.ds(i * DEG, DEG)] = e / jnp.sum(e)

    pltpu.sync_copy(out_v, out_hbm.at[pl.ds(ebase, _EDGE_PER)])


@functools.cache
def _sc_softmax_kernel():
    return pl.kernel(
        _sc_softmax_body,
        out_type=jax.ShapeDtypeStruct((N_EDGES,), jnp.float32),
        mesh=plsc.VectorSubcoreMesh(
            core_axis_name="c", subcore_axis_name="s",
            num_cores=_NC, num_subcores=_NS),
        scratch_types=[
            pltpu.VMEM((N_TASK,), jnp.float32),
            pltpu.VMEM((_AG_PER + DEG,), jnp.float32),
            pltpu.VMEM((_EDGE_PER,), jnp.int32),
            pltpu.VMEM((_EDGE_PER,), jnp.float32),
        ],
        compiler_params=pltpu.CompilerParams(needs_layout_passes=False),
    )


def kernel(nf, edge_index, node_type, W_att):
    w2 = W_att.reshape(2, EMBED_DIM)
    a_full, b_full = _tc_project(nf, node_type, w2)
    flat = _sc_softmax_kernel()(a_full, b_full, edge_index)
    return flat.reshape(N_AG, DEG)


# speedup vs baseline: 1.0729x; 1.0117x over previous
import functools

import jax
import jax.numpy as jnp
from jax import lax
from jax.experimental import pallas as pl
from jax.experimental.pallas import tpu as pltpu
from jax.experimental.pallas import tpu_sc as plsc

EMBED_DIM = 256
N_AG = 10000
N_TASK = 10000
N_NODES = N_AG + N_TASK
DEG = 16
N_EDGES = N_AG * DEG
FIN_TASK_TYPE = 3
NEG_SLOPE = 0.01

_NC = 2
_NS = 16
_NW = _NC * _NS
_AG_PER = 320
_EDGE_PER = _AG_PER * DEG

_TC_ROWS = 8192


def _tc_project_body(nf_ref, w_ref, t_ref, a_ref, b_ref):
    x = nf_ref[...]
    w = w_ref[...]
    p = lax.dot_general(
        w, x, dimension_numbers=(((1,), (1,)), ((), ())),
        preferred_element_type=jnp.float32)
    fin = t_ref[...] == FIN_TASK_TYPE
    a_ref[...] = jnp.where(fin, -jnp.inf, p[0, :])
    b_ref[...] = p[1, :]


def _tc_project(nf, node_type, w2):
    return pl.pallas_call(
        _tc_project_body,
        grid=(pl.cdiv(N_NODES, _TC_ROWS),),
        in_specs=[
            pl.BlockSpec((_TC_ROWS, EMBED_DIM), lambda i: (i, 0)),
            pl.BlockSpec((2, EMBED_DIM), lambda i: (0, 0)),
            pl.BlockSpec((_TC_ROWS,), lambda i: (i,)),
        ],
        out_specs=[
            pl.BlockSpec((_TC_ROWS,), lambda i: (i,)),
            pl.BlockSpec((_TC_ROWS,), lambda i: (i,)),
        ],
        out_shape=[
            jax.ShapeDtypeStruct((N_NODES,), jnp.float32),
            jax.ShapeDtypeStruct((N_NODES,), jnp.float32),
        ],
    )(nf, w2, node_type)


def _sc_softmax_body(a_hbm, b_hbm, ei_hbm, out_hbm, a_v, b_v, idx_v, out_v):
    wid = lax.axis_index("s") * _NC + lax.axis_index("c")
    base = jnp.minimum(wid * _AG_PER, N_AG - _AG_PER)
    ebase = base * DEG
    pltpu.sync_copy(a_hbm.at[pl.ds(N_AG, N_TASK)], a_v)
    pltpu.sync_copy(b_hbm.at[pl.ds(base, _AG_PER)], b_v.at[pl.ds(0, _AG_PER)])
    pltpu.sync_copy(ei_hbm.at[0].at[pl.ds(ebase, _EDGE_PER)], idx_v)

    @plsc.parallel_loop(0, _AG_PER, unroll=4)
    def _(i):
        idx = idx_v[pl.ds(i * DEG, DEG)] - N_AG
        av = plsc.load_gather(a_v, [idx])
        bg = b_v[pl.ds(i, DEG)][0]
        s = av + bg
        s = jnp.where(s >= 0, s, s * NEG_SLOPE)
        m = jnp.max(s)
        e = jnp.exp(s - m)
        out_v[pl.ds(i * DEG, DEG)] = e / jnp.sum(e)

    pltpu.sync_copy(out_v, out_hbm.at[pl.ds(ebase, _EDGE_PER)])


@functools.cache
def _sc_softmax_kernel():
    return pl.kernel(
        _sc_softmax_body,
        out_type=jax.ShapeDtypeStruct((N_EDGES,), jnp.float32),
        mesh=plsc.VectorSubcoreMesh(
            core_axis_name="c", subcore_axis_name="s",
            num_cores=_NC, num_subcores=_NS),
        scratch_types=[
            pltpu.VMEM((N_TASK,), jnp.float32),
            pltpu.VMEM((_AG_PER + DEG,), jnp.float32),
            pltpu.VMEM((_EDGE_PER,), jnp.int32),
            pltpu.VMEM((_EDGE_PER,), jnp.float32),
        ],
        compiler_params=pltpu.CompilerParams(needs_layout_passes=False),
    )


def kernel(nf, edge_index, node_type, W_att):
    w2 = W_att.reshape(2, EMBED_DIM)
    a_full, b_full = _tc_project(nf, node_type, w2)
    flat = _sc_softmax_kernel()(a_full, b_full, edge_index)
    return flat.reshape(N_AG, DEG)


# speedup vs baseline: 1.0785x; 1.0052x over previous
import functools

import jax
import jax.numpy as jnp
from jax import lax
from jax.experimental import pallas as pl
from jax.experimental.pallas import tpu as pltpu
from jax.experimental.pallas import tpu_sc as plsc

EMBED_DIM = 256
N_AG = 10000
N_TASK = 10000
N_NODES = N_AG + N_TASK
DEG = 16
N_EDGES = N_AG * DEG
FIN_TASK_TYPE = 3
NEG_SLOPE = 0.01

_NC = 2
_NS = 16
_NW = _NC * _NS
_AG_PER = 320
_EDGE_PER = _AG_PER * DEG

_TC_ROWS = 8192


def _tc_project_body(nf_ref, w_ref, t_ref, a_ref, b_ref):
    x = nf_ref[...]
    w = w_ref[...]
    p = lax.dot_general(
        w, x, dimension_numbers=(((1,), (1,)), ((), ())),
        preferred_element_type=jnp.float32)
    fin = t_ref[...] == FIN_TASK_TYPE
    a_ref[...] = jnp.where(fin, -jnp.inf, p[0, :])
    b_ref[...] = p[1, :]


def _tc_project(nf, node_type, w2):
    return pl.pallas_call(
        _tc_project_body,
        grid=(pl.cdiv(N_NODES, _TC_ROWS),),
        in_specs=[
            pl.BlockSpec((_TC_ROWS, EMBED_DIM), lambda i: (i, 0)),
            pl.BlockSpec((2, EMBED_DIM), lambda i: (0, 0)),
            pl.BlockSpec((_TC_ROWS,), lambda i: (i,)),
        ],
        out_specs=[
            pl.BlockSpec((_TC_ROWS,), lambda i: (i,)),
            pl.BlockSpec((_TC_ROWS,), lambda i: (i,)),
        ],
        out_shape=[
            jax.ShapeDtypeStruct((N_NODES,), jnp.float32),
            jax.ShapeDtypeStruct((N_NODES,), jnp.float32),
        ],
    )(nf, w2, node_type)


def _sc_softmax_body(a_hbm, b_hbm, ei_hbm, out_hbm, a_v, b_v, idx_v, out_v):
    wid = lax.axis_index("s") * _NC + lax.axis_index("c")
    base = jnp.minimum(wid * _AG_PER, N_AG - _AG_PER)
    ebase = base * DEG
    pltpu.sync_copy(a_hbm.at[pl.ds(N_AG, N_TASK)], a_v)
    pltpu.sync_copy(b_hbm.at[pl.ds(base, _AG_PER)], b_v.at[pl.ds(0, _AG_PER)])
    pltpu.sync_copy(ei_hbm.at[0].at[pl.ds(ebase, _EDGE_PER)], idx_v)

    @plsc.parallel_loop(0, _AG_PER, unroll=4)
    def _(i):
        idx = idx_v[pl.ds(i * DEG, DEG)] - N_AG
        av = plsc.load_gather(a_v, [idx])
        bg = b_v[pl.ds(i, DEG)][0]
        s = av + bg
        s = jnp.where(s >= 0, s, s * NEG_SLOPE)
        m = jnp.max(s)
        e = jnp.exp(s - m)
        out_v[i] = e / jnp.sum(e)

    pltpu.sync_copy(out_v, out_hbm.at[pl.ds(base, _AG_PER)])


@functools.cache
def _sc_softmax_kernel():
    return pl.kernel(
        _sc_softmax_body,
        out_type=jax.ShapeDtypeStruct((N_AG, DEG), jnp.float32),
        mesh=plsc.VectorSubcoreMesh(
            core_axis_name="c", subcore_axis_name="s",
            num_cores=_NC, num_subcores=_NS),
        scratch_types=[
            pltpu.VMEM((N_TASK,), jnp.float32),
            pltpu.VMEM((_AG_PER + DEG,), jnp.float32),
            pltpu.VMEM((_EDGE_PER,), jnp.int32),
            pltpu.VMEM((_AG_PER, DEG), jnp.float32),
        ],
        compiler_params=pltpu.CompilerParams(needs_layout_passes=False),
    )


def kernel(nf, edge_index, node_type, W_att):
    w2 = W_att.reshape(2, EMBED_DIM)
    a_full, b_full = _tc_project(nf, node_type, w2)
    return _sc_softmax_kernel()(a_full, b_full, edge_index)


# speedup vs baseline: 1.0976x; 1.0177x over previous
import functools

import jax
import jax.numpy as jnp
from jax import lax
from jax.experimental import pallas as pl
from jax.experimental.pallas import tpu as pltpu
from jax.experimental.pallas import tpu_sc as plsc

EMBED_DIM = 256
N_AG = 10000
N_TASK = 10000
N_NODES = N_AG + N_TASK
DEG = 16
N_EDGES = N_AG * DEG
FIN_TASK_TYPE = 3
NEG_SLOPE = 0.01

_NC = 2
_NS = 16
_NW = _NC * _NS
_AG_PER = 320
_EDGE_PER = _AG_PER * DEG

_TC_ROWS = 8192


def _tc_project_body(nf_ref, w_ref, t_ref, a_ref, b_ref):
    x = nf_ref[...]
    w = w_ref[...]
    p = lax.dot_general(
        w, x, dimension_numbers=(((1,), (1,)), ((), ())),
        preferred_element_type=jnp.float32)
    fin = t_ref[...] == FIN_TASK_TYPE
    a_ref[...] = jnp.where(fin, -jnp.inf, p[0, :])
    b_ref[...] = p[1, :]


def _tc_project(nf, node_type, w2):
    return pl.pallas_call(
        _tc_project_body,
        grid=(pl.cdiv(N_NODES, _TC_ROWS),),
        in_specs=[
            pl.BlockSpec((_TC_ROWS, EMBED_DIM), lambda i: (i, 0)),
            pl.BlockSpec((2, EMBED_DIM), lambda i: (0, 0)),
            pl.BlockSpec((_TC_ROWS,), lambda i: (i,)),
        ],
        out_specs=[
            pl.BlockSpec((_TC_ROWS,), lambda i: (i,)),
            pl.BlockSpec((_TC_ROWS,), lambda i: (i,)),
        ],
        out_shape=[
            jax.ShapeDtypeStruct((N_NODES,), jnp.float32),
            jax.ShapeDtypeStruct((N_NODES,), jnp.float32),
        ],
    )(nf, w2, node_type)


def _sc_softmax_body(a_hbm, b_hbm, ei_hbm, out_hbm, a_v, b_v, idx_v, out_v,
                     sem_a, sem_b, sem_i):
    wid = lax.axis_index("s") * _NC + lax.axis_index("c")
    base = jnp.minimum(wid * _AG_PER, N_AG - _AG_PER)
    ebase = base * DEG
    c_a = pltpu.async_copy(a_hbm.at[pl.ds(N_AG, N_TASK)], a_v, sem_a)
    c_b = pltpu.async_copy(b_hbm.at[pl.ds(base, _AG_PER)],
                           b_v.at[pl.ds(0, _AG_PER)], sem_b)
    c_i = pltpu.async_copy(ei_hbm.at[0].at[pl.ds(ebase, _EDGE_PER)], idx_v,
                           sem_i)
    c_a.wait()
    c_b.wait()
    c_i.wait()

    @plsc.parallel_loop(0, _AG_PER, unroll=8)
    def _(i):
        idx = idx_v[pl.ds(i * DEG, DEG)] - N_AG
        av = plsc.load_gather(a_v, [idx])
        bg = b_v[pl.ds(i, DEG)][0]
        s = av + bg
        s = jnp.where(s >= 0, s, s * NEG_SLOPE)
        m = jnp.max(s)
        e = jnp.exp(s - m)
        out_v[pl.ds(i * DEG, DEG)] = e / jnp.sum(e)

    pltpu.sync_copy(out_v, out_hbm.at[pl.ds(ebase, _EDGE_PER)])


@functools.cache
def _sc_softmax_kernel():
    return pl.kernel(
        _sc_softmax_body,
        out_type=jax.ShapeDtypeStruct((N_EDGES,), jnp.float32),
        mesh=plsc.VectorSubcoreMesh(
            core_axis_name="c", subcore_axis_name="s",
            num_cores=_NC, num_subcores=_NS),
        scratch_types=[
            pltpu.VMEM((N_TASK,), jnp.float32),
            pltpu.VMEM((_AG_PER + DEG,), jnp.float32),
            pltpu.VMEM((_EDGE_PER,), jnp.int32),
            pltpu.VMEM((_EDGE_PER,), jnp.float32),
            pltpu.SemaphoreType.DMA,
            pltpu.SemaphoreType.DMA,
            pltpu.SemaphoreType.DMA,
        ],
        compiler_params=pltpu.CompilerParams(needs_layout_passes=False),
    )


def kernel(nf, edge_index, node_type, W_att):
    w2 = W_att.reshape(2, EMBED_DIM)
    a_full, b_full = _tc_project(nf, node_type, w2)
    flat = _sc_softmax_kernel()(a_full, b_full, edge_index)
    return flat.reshape(N_AG, DEG)
